# baseline (speedup 1.0000x reference)
import functools

import jax
import jax.numpy as jnp
from jax import lax
from jax.experimental import pallas as pl
from jax.experimental.pallas import tpu as pltpu
from jax.experimental.pallas import tpu_sc as plsc

_M = 0.25
_GAMMA = 64.0
_B = 1024
_V = 100000
_CB = 2560
_NCB = -(-_V // _CB)
_NEG = -1e30


_NW = 32
_BPW = _B // _NW


def _sc_gather_body(tab_hbm, lab_hbm, out_hbm, lab_v, idx_v, g_v, sem):
    c = lax.axis_index("c")
    s = lax.axis_index("s")
    wid = s * 2 + c
    base = wid * _BPW
    pltpu.sync_copy(lab_hbm.at[pl.ds(base, _BPW)], lab_v)
    for j in range(_BPW // 16):
        sl = pl.ds(j * 16, 16)
        bvec = lax.iota(jnp.int32, 16) + (base + j * 16)
        idx_v[sl] = bvec * _V + lab_v[sl]
    pltpu.async_copy(tab_hbm.at[idx_v], g_v, sem).wait()
    pltpu.sync_copy(g_v, out_hbm.at[pl.ds(base, _BPW)])


@functools.lru_cache(maxsize=1)
def _sc_gather():
    return pl.kernel(
        _sc_gather_body,
        out_type=jax.ShapeDtypeStruct((_B,), jnp.float32),
        mesh=plsc.VectorSubcoreMesh(core_axis_name="c", subcore_axis_name="s"),
        scratch_types=[
            pltpu.VMEM((_BPW,), jnp.int32),
            pltpu.VMEM((_BPW,), jnp.int32),
            pltpu.VMEM((_BPW,), jnp.float32),
            pltpu.SemaphoreType.DMA,
        ],
    )


_G2 = _GAMMA * 1.4426950408889634
_LN2 = 0.6931471805599453


def _tc_body(lab_ref, g_ref, x_ref, out_ref, m_scr, s_scr):
    cb = pl.program_id(0)

    @pl.when(cb == 0)
    def _init():
        m_scr[...] = jnp.full((_B, 1), _NEG, dtype=jnp.float32)
        s_scr[...] = jnp.zeros((_B, 1), dtype=jnp.float32)

    x = x_ref[...]
    l2 = (_G2 * jnp.maximum(x + _M, 0.0)) * (x - _M)
    l2 = lax.cond(
        cb == _NCB - 1,
        lambda v: jnp.where(
            lax.broadcasted_iota(jnp.int32, (_B, _CB), 1) >= _V - cb * _CB,
            _NEG,
            v,
        ),
        lambda v: v,
        l2,
    )

    bm = jnp.max(l2, axis=1, keepdims=True)
    m_old = m_scr[...]
    m_new = jnp.maximum(m_old, bm)
    p = jnp.exp2(l2 - m_new)
    bs = jnp.sum(p, axis=1, keepdims=True)
    g = g_ref[...]
    lab_loc = lab_ref[...] - cb * _CB
    in_blk = (lab_loc >= 0) & (lab_loc < _CB)
    lw2 = (_G2 * jnp.maximum(g + _M, 0.0)) * (g - _M)
    corr = jnp.where(in_blk, jnp.exp2(lw2 - m_new), 0.0)
    s_scr[...] = s_scr[...] * jnp.exp2(m_old - m_new) + (bs - corr)
    m_scr[...] = m_new

    @pl.when(cb == _NCB - 1)
    def _finish():
        gg = g_ref[...]
        lc2 = (_G2 * jnp.maximum(1.0 + _M - gg, 0.0)) * (gg - (1.0 - _M))
        m2w = m_scr[...]
        mx2 = jnp.maximum(m2w, lc2)
        sm = s_scr[...] * jnp.exp2(m2w - mx2) + jnp.exp2(lc2 - mx2)
        sm = jnp.maximum(sm, 1e-37)
        nll2 = mx2 + jnp.log2(sm) - lc2
        out_ref[0, 0] = jnp.sum(nll2) * (_LN2 / _B)


_tc_loss = pl.pallas_call(
    _tc_body,
    grid=(_NCB,),
    in_specs=[
        pl.BlockSpec((_B, 1), lambda cb: (0, 0)),
        pl.BlockSpec((_B, 1), lambda cb: (0, 0)),
        pl.BlockSpec((_B, _CB), lambda cb: (0, cb)),
    ],
    out_specs=pl.BlockSpec(memory_space=pltpu.SMEM),
    out_shape=jax.ShapeDtypeStruct((1, 1), jnp.float32),
    scratch_shapes=[
        pltpu.VMEM((_B, 1), jnp.float32),
        pltpu.VMEM((_B, 1), jnp.float32),
    ],
    compiler_params=pltpu.CompilerParams(
        dimension_semantics=("arbitrary",),
    ),
)


def kernel(inp, label):
    tab = inp.reshape(_B * _V)
    g = _sc_gather()(tab, label)
    out = _tc_loss(label.reshape(_B, 1), g.reshape(_B, 1), inp)
    return out[0, 0]


# speedup vs baseline: 1.1875x; 1.1875x over previous
import functools

import jax
import jax.numpy as jnp
from jax import lax
from jax.experimental import pallas as pl
from jax.experimental.pallas import tpu as pltpu
from jax.experimental.pallas import tpu_sc as plsc

_M = 0.25
_GAMMA = 64.0
_B = 1024
_V = 100000
_CB = 4096
_NCB = 24
_TAIL = _V - _NCB * _CB
_NEG = -1e30
_G2 = _GAMMA * 1.4426950408889634
_LN2 = 0.6931471805599453


_NW = 32
_BPW = _B // _NW


def _sc_gather_body(tab_hbm, lab_hbm, out_hbm, lab_v, idx_v, g_v, sem):
    c = lax.axis_index("c")
    s = lax.axis_index("s")
    wid = s * 2 + c
    base = wid * _BPW
    pltpu.sync_copy(lab_hbm.at[pl.ds(base, _BPW)], lab_v)
    for j in range(_BPW // 16):
        sl = pl.ds(j * 16, 16)
        bvec = lax.iota(jnp.int32, 16) + (base + j * 16)
        idx_v[sl] = bvec * _V + lab_v[sl]
    pltpu.async_copy(tab_hbm.at[idx_v], g_v, sem).wait()
    pltpu.sync_copy(g_v, out_hbm.at[pl.ds(base, _BPW)])


@functools.lru_cache(maxsize=1)
def _sc_gather():
    return pl.kernel(
        _sc_gather_body,
        out_type=jax.ShapeDtypeStruct((_B,), jnp.float32),
        mesh=plsc.VectorSubcoreMesh(core_axis_name="c", subcore_axis_name="s"),
        scratch_types=[
            pltpu.VMEM((_BPW,), jnp.int32),
            pltpu.VMEM((_BPW,), jnp.int32),
            pltpu.VMEM((_BPW,), jnp.float32),
            pltpu.SemaphoreType.DMA,
        ],
    )


def _wrong_logit2(x):
    return (_G2 * jnp.maximum(x + _M, 0.0)) * (x - _M)


def _tc1_body(lab_ref, g_ref, x_ref, mo_ref, so_ref, m_scr, s_scr):
    cb = pl.program_id(0)

    @pl.when(cb == 0)
    def _init():
        m_scr[...] = jnp.full((_B, 1), _NEG, dtype=jnp.float32)
        s_scr[...] = jnp.zeros((_B, 1), dtype=jnp.float32)

    x = x_ref[...]
    l2 = _wrong_logit2(x)
    bm = jnp.max(l2, axis=1, keepdims=True)
    m_old = m_scr[...]
    m_new = jnp.maximum(m_old, bm)
    p = jnp.exp2(l2 - m_new)
    bs = jnp.sum(p, axis=1, keepdims=True)
    lab_loc = lab_ref[...] - cb * _CB
    in_blk = (lab_loc >= 0) & (lab_loc < _CB)
    lw2 = _wrong_logit2(g_ref[...])
    corr = jnp.where(in_blk, jnp.exp2(lw2 - m_new), 0.0)
    s_scr[...] = s_scr[...] * jnp.exp2(m_old - m_new) + (bs - corr)
    m_scr[...] = m_new

    @pl.when(cb == _NCB - 1)
    def _out():
        mo_ref[...] = m_scr[...]
        so_ref[...] = s_scr[...]


def _tc2_body(lab_ref, g_ref, mi_ref, si_ref, xt_ref, out_ref):
    x = xt_ref[...]
    l2 = _wrong_logit2(x)
    bm = jnp.max(l2, axis=1, keepdims=True)
    m_old = mi_ref[...]
    m_new = jnp.maximum(m_old, bm)
    p = jnp.exp2(l2 - m_new)
    bs = jnp.sum(p, axis=1, keepdims=True)
    g = g_ref[...]
    lab_loc = lab_ref[...] - _NCB * _CB
    in_blk = lab_loc >= 0
    lw2 = _wrong_logit2(g)
    corr = jnp.where(in_blk, jnp.exp2(lw2 - m_new), 0.0)
    sw = si_ref[...] * jnp.exp2(m_old - m_new) + (bs - corr)
    m2w = m_new

    lc2 = (_G2 * jnp.maximum(1.0 + _M - g, 0.0)) * (g - (1.0 - _M))
    mx2 = jnp.maximum(m2w, lc2)
    sm = sw * jnp.exp2(m2w - mx2) + jnp.exp2(lc2 - mx2)
    sm = jnp.maximum(sm, 1e-37)
    nll2 = mx2 + jnp.log2(sm) - lc2
    out_ref[0, 0] = jnp.sum(nll2) * (_LN2 / _B)


def _build_tc(interpret=False):
    tc1 = pl.pallas_call(
        _tc1_body,
        grid=(_NCB,),
        in_specs=[
            pl.BlockSpec((_B, 1), lambda cb: (0, 0)),
            pl.BlockSpec((_B, 1), lambda cb: (0, 0)),
            pl.BlockSpec((_B, _CB), lambda cb: (0, cb)),
        ],
        out_specs=[
            pl.BlockSpec((_B, 1), lambda cb: (0, 0)),
            pl.BlockSpec((_B, 1), lambda cb: (0, 0)),
        ],
        out_shape=[
            jax.ShapeDtypeStruct((_B, 1), jnp.float32),
            jax.ShapeDtypeStruct((_B, 1), jnp.float32),
        ],
        scratch_shapes=[
            pltpu.VMEM((_B, 1), jnp.float32),
            pltpu.VMEM((_B, 1), jnp.float32),
        ],
        compiler_params=pltpu.CompilerParams(
            dimension_semantics=("arbitrary",),
        ),
        interpret=interpret,
    )
    tc2 = pl.pallas_call(
        _tc2_body,
        out_specs=pl.BlockSpec(memory_space=pltpu.SMEM),
        out_shape=jax.ShapeDtypeStruct((1, 1), jnp.float32),
        interpret=interpret,
    )

    def run(label2d, g2d, inp):
        m, s = tc1(label2d, g2d, inp)
        xt = lax.slice(inp, (0, _NCB * _CB), (_B, _V))
        return tc2(label2d, g2d, m, s, xt)

    return run


_tc_loss = _build_tc()


def kernel(inp, label):
    tab = inp.reshape(_B * _V)
    g = _sc_gather()(tab, label)
    out = _tc_loss(label.reshape(_B, 1), g.reshape(_B, 1), inp)
    return out[0, 0]
